# baseline (speedup 1.0000x reference)
import functools

import jax
import jax.numpy as jnp
from jax import lax
from jax.experimental import pallas as pl
from jax.experimental.pallas import tpu as pltpu
from jax.experimental.pallas import tpu_sc as plsc

N = 10000
E = 320000
NSLICE = 2
ES = E // NSLICE
DX = 48
DH = 128
NC, NS = 2, 16
NW = NC * NS
EPW = ES // NW
CH = 200
NCHUNK = EPW // CH
SCH = 40
NSCH = EPW // SCH
CC = 2000
NP = 10240
STRIPE = NP // NS
_F32 = jnp.float32
_PREC = lax.Precision.DEFAULT


def _mesh():
    return plsc.VectorSubcoreMesh(core_axis_name="c", subcore_axis_name="s")


_SC_PARAMS = dict(
    mesh=_mesh(),
    compiler_params=pltpu.CompilerParams(needs_layout_passes=False),
)


def _count_body(col_hbm, cnt_hbm, colv, cnt_v):
    wid = lax.axis_index("s") * NC + lax.axis_index("c")
    base = wid * (E // NW)
    zeros16 = jnp.zeros((16,), _F32)
    ones16 = jnp.ones((16,), _F32)

    def zbody(i, carry):
        cnt_v[pl.ds(i * 16, 16)] = zeros16
        return carry

    lax.fori_loop(0, N // 16, zbody, 0)

    def body(k, carry):
        pltpu.sync_copy(col_hbm.at[pl.ds(base + k * CC, CC)], colv)

        def cbody(j, c):
            cvec = colv[pl.ds(j * 16, 16)]
            plsc.addupdate_scatter(cnt_v, [cvec], ones16)
            return c

        lax.fori_loop(0, CC // 16, cbody, 0)
        return carry

    lax.fori_loop(0, (E // NW) // CC, body, 0)
    pltpu.sync_copy(cnt_v, cnt_hbm.at[pl.ds(wid * N, N)])


@functools.cache
def _count_kernel():
    return pl.kernel(
        _count_body,
        out_type=jax.ShapeDtypeStruct((NW * N,), _F32),
        scratch_types=[
            pltpu.VMEM((CC,), jnp.int32),
            pltpu.VMEM((N,), _F32),
        ],
        **_SC_PARAMS,
    )


def _gather_body(sbase, xw_hbm, row_hbm, out_hbm,
                 idx0, idx1, rows0, rows1, sem_i0, sem_i1, sem_g0, sem_g1):
    wid = lax.axis_index("s") * NC + lax.axis_index("c")
    base = sbase + wid * EPW
    obase = wid * EPW
    sem_i = (sem_i0, sem_i1)
    sem_g = (sem_g0, sem_g1)
    idx_v = (idx0, idx1)
    rows_v = (rows0, rows1)

    def load(k, b):
        pltpu.async_copy(row_hbm.at[pl.ds(base + k * CH, CH)], idx_v[b], sem_i[b])

    def wait_load(k, b):
        pltpu.make_async_copy(row_hbm.at[pl.ds(base + k * CH, CH)], idx_v[b],
                              sem_i[b]).wait()

    load(0, 0)

    def step(k, b):
        wait_load(k, b)
        pltpu.async_copy(xw_hbm.at[idx_v[b]], rows_v[b], sem_g[b])

        @pl.when(k >= 1)
        def _():
            po = obase + (k - 1) * CH
            pltpu.make_async_copy(xw_hbm.at[idx_v[1 - b]], rows_v[1 - b],
                                  sem_g[1 - b]).wait()
            pltpu.sync_copy(rows_v[1 - b], out_hbm.at[pl.ds(po, CH)])

        @pl.when(k + 1 <= NCHUNK - 1)
        def _():
            load(k + 1, 1 - b)

    def pair(g, carry):
        step(2 * g, 0)
        step(2 * g + 1, 1)
        return carry

    lax.fori_loop(0, NCHUNK // 2, pair, 0)
    k_last = NCHUNK - 1
    wait_load(k_last, 0)
    pltpu.async_copy(xw_hbm.at[idx_v[0]], rows_v[0], sem_g[0])
    pltpu.make_async_copy(xw_hbm.at[idx_v[1]], rows_v[1], sem_g[1]).wait()
    pltpu.sync_copy(rows_v[1], out_hbm.at[pl.ds(obase + (k_last - 1) * CH, CH)])
    pltpu.make_async_copy(xw_hbm.at[idx_v[0]], rows_v[0], sem_g[0]).wait()
    pltpu.sync_copy(rows_v[0], out_hbm.at[pl.ds(obase + k_last * CH, CH)])


@functools.cache
def _gather_kernel(s):
    return pl.kernel(
        functools.partial(_gather_body, s * ES),
        out_type=jax.ShapeDtypeStruct((ES, DH), _F32),
        scratch_types=[
            pltpu.VMEM((CH,), jnp.int32),
            pltpu.VMEM((CH,), jnp.int32),
            pltpu.VMEM((CH, DH), _F32),
            pltpu.VMEM((CH, DH), _F32),
            pltpu.SemaphoreType.DMA,
            pltpu.SemaphoreType.DMA,
            pltpu.SemaphoreType.DMA,
            pltpu.SemaphoreType.DMA,
        ],
        **_SC_PARAMS,
    )


def _scatter_body(sbase, y_hbm, col_hbm, z128_hbm, out_s, tab_s, sidx0, sidx1,
                  ybuf0, ybuf1, sem_d0, sem_d1, sem_a0, sem_a1):
    cid = lax.axis_index("c")
    sid = lax.axis_index("s")
    wid = sid * NC + cid
    srow = sid * STRIPE
    sem_d = (sem_d0, sem_d1)
    sem_a = (sem_a0, sem_a1)
    idx_v = (sidx0, sidx1)
    ybuf = (ybuf0, ybuf1)
    pltpu.sync_copy(z128_hbm.at[pl.ds(srow, STRIPE)], tab_s.at[pl.ds(srow, STRIPE)])
    plsc.subcore_barrier()
    base = wid * EPW
    cbase = sbase + wid * EPW

    def load(k, b):
        pltpu.async_copy(col_hbm.at[pl.ds(cbase + k * SCH, SCH)], idx_v[b], sem_d[b])
        pltpu.async_copy(y_hbm.at[pl.ds(base + k * SCH, SCH)], ybuf[b], sem_d[b])

    def wait_load(k, b):
        pltpu.make_async_copy(col_hbm.at[pl.ds(cbase + k * SCH, SCH)], idx_v[b],
                              sem_d[b]).wait()
        pltpu.make_async_copy(y_hbm.at[pl.ds(base + k * SCH, SCH)], ybuf[b],
                              sem_d[b]).wait()

    def wait_add(b):
        pltpu.make_async_copy(ybuf[b], tab_s.at[idx_v[b]], sem_a[b]).wait()

    load(0, 0)
    load(1, 1)

    def step(k, b):
        wait_load(k, b)
        pltpu.async_copy(ybuf[b], tab_s.at[idx_v[b]], sem_a[b], add=True)

        @pl.when(jnp.logical_and(k >= 1, k + 1 <= NSCH - 1))
        def _():
            wait_add(1 - b)
            load(k + 1, 1 - b)

    def pair(g, carry):
        step(2 * g, 0)
        step(2 * g + 1, 1)
        return carry

    lax.fori_loop(0, NSCH // 2, pair, 0)
    wait_load(NSCH - 1, 0)
    pltpu.async_copy(ybuf[0], tab_s.at[idx_v[0]], sem_a[0], add=True)
    wait_add(1)
    wait_add(0)
    plsc.subcore_barrier()
    pltpu.sync_copy(tab_s.at[pl.ds(srow, STRIPE)],
                    out_s.at[pl.ds(cid * NP + srow, STRIPE)])


@functools.cache
def _scatter_kernel(s):
    return pl.kernel(
        functools.partial(_scatter_body, s * ES),
        out_type=jax.ShapeDtypeStruct((NC * NP, DH), _F32),
        scratch_types=[
            pltpu.VMEM_SHARED((NP, DH), _F32),
            pltpu.VMEM((SCH,), jnp.int32),
            pltpu.VMEM((SCH,), jnp.int32),
            pltpu.VMEM((SCH, DH), _F32),
            pltpu.VMEM((SCH, DH), _F32),
            pltpu.SemaphoreType.DMA,
            pltpu.SemaphoreType.DMA,
            pltpu.SemaphoreType.DMA,
            pltpu.SemaphoreType.DMA,
        ],
        **_SC_PARAMS,
    )


BE = 3200


def _xw_body(x_ref, a_ref, bias_ref, o_ref):
    o_ref[...] = (jnp.dot(x_ref[...], a_ref[...], preferred_element_type=_F32,
                          precision=_PREC) + bias_ref[...])


def _mlp1_body(xwg_ref, ea_ref, b_ref, t_ref, stat_ref):
    t = jnp.dot(ea_ref[...], b_ref[...], preferred_element_type=_F32,
                precision=_PREC)
    t = t + xwg_ref[...]
    t_ref[...] = t
    srow = jnp.sum(t, axis=0, keepdims=True)
    sqrow = jnp.sum(t * t, axis=0, keepdims=True)
    upd = jnp.concatenate([srow, sqrow, jnp.zeros((6, DH), _F32)], axis=0)

    @pl.when(pl.program_id(0) == 0)
    def _():
        stat_ref[...] = jnp.zeros_like(stat_ref)

    stat_ref[...] += upd


def _mlp1b_body(t_ref, stat_a_ref, stat_b_ref, g_ref, be_ref, w_ref, bias_ref,
                y_ref):
    stat = stat_a_ref[...] + stat_b_ref[...]
    mean = stat[0:1, :] * (1.0 / E)
    var = stat[1:2, :] * (1.0 / E) - mean * mean
    scale = g_ref[...] * lax.rsqrt(var + 1e-5)
    shift = be_ref[...] - mean * scale
    tn = jnp.maximum(t_ref[...] * scale + shift, 0.0)
    y_ref[...] = (jnp.dot(tn, w_ref[...], preferred_element_type=_F32,
                          precision=_PREC) + bias_ref[...])


def _node_body(sa_ref, sb_ref, cp_ref, rones_ref, x_ref, a_ref, b_ref, b2a_ref,
               g_ref, be_ref, w_ref, b2b_ref, out_ref):
    ssum = (sa_ref[0:N, :] + sa_ref[NP:NP + N, :]
            + sb_ref[0:N, :] + sb_ref[NP:NP + N, :])
    cnt = lax.dot_general(cp_ref[...], rones_ref[...], (((0,), (0,)), ((), ())),
                          preferred_element_type=_F32,
                          precision=lax.Precision.HIGHEST)
    agg = jnp.where(cnt > 0.0, ssum / jnp.maximum(cnt, 1.0), 0.0)
    h = (jnp.dot(x_ref[...], a_ref[...], preferred_element_type=_F32,
                 precision=_PREC)
         + jnp.dot(agg, b_ref[...], preferred_element_type=_F32, precision=_PREC)
         + b2a_ref[...])
    mean = jnp.mean(h, axis=0, keepdims=True)
    var = jnp.mean((h - mean) ** 2, axis=0, keepdims=True)
    hn = jnp.maximum((h - mean) * lax.rsqrt(var + 1e-5) * g_ref[...] + be_ref[...],
                     0.0)
    out_ref[...] = (jnp.dot(hn, w_ref[...], preferred_element_type=_F32,
                            precision=_PREC) + b2b_ref[...])


def _xw(x, a1, bias):
    return pl.pallas_call(
        _xw_body,
        out_shape=jax.ShapeDtypeStruct((N, DH), _F32),
    )(x, a1, bias)


def _mlp1(xwg, ea, b1):
    return pl.pallas_call(
        _mlp1_body,
        grid=(ES // BE,),
        in_specs=[
            pl.BlockSpec((BE, DH), lambda i: (i, 0)),
            pl.BlockSpec((BE, DH), lambda i: (i, 0)),
            pl.BlockSpec((DH, DH), lambda i: (0, 0)),
        ],
        out_specs=[
            pl.BlockSpec((BE, DH), lambda i: (i, 0)),
            pl.BlockSpec((8, DH), lambda i: (0, 0)),
        ],
        out_shape=[
            jax.ShapeDtypeStruct((ES, DH), _F32),
            jax.ShapeDtypeStruct((8, DH), _F32),
        ],
    )(xwg, ea, b1)


def _mlp1b(t, stats_a, stats_b, g, be, w, bias):
    return pl.pallas_call(
        _mlp1b_body,
        grid=(ES // BE,),
        in_specs=[
            pl.BlockSpec((BE, DH), lambda i: (i, 0)),
            pl.BlockSpec((8, DH), lambda i: (0, 0)),
            pl.BlockSpec((8, DH), lambda i: (0, 0)),
            pl.BlockSpec((1, DH), lambda i: (0, 0)),
            pl.BlockSpec((1, DH), lambda i: (0, 0)),
            pl.BlockSpec((DH, DH), lambda i: (0, 0)),
            pl.BlockSpec((1, DH), lambda i: (0, 0)),
        ],
        out_specs=pl.BlockSpec((BE, DH), lambda i: (i, 0)),
        out_shape=jax.ShapeDtypeStruct((ES, DH), _F32),
    )(t, stats_a, stats_b, g, be, w, bias)


def _node_mlp(sa, sb, cp, rones, x, a2, b2, b2a, g2, be2, w2b, b2b):
    return pl.pallas_call(
        _node_body,
        out_shape=jax.ShapeDtypeStruct((N, DH), _F32),
    )(sa, sb, cp, rones, x, a2, b2, b2a, g2, be2, w2b, b2b)


def kernel(x, edge_index, edge_attr, u, batch, W1a, b1a, g1, be1, W1b, b1b,
           W2a, b2a, g2, be2, W2b, b2b):
    row = edge_index[0]
    col = edge_index[1]
    cntp = _count_kernel()(col)
    xw = _xw(x, W1a[:DX], b1a.reshape(1, DH))
    xwg = [_gather_kernel(s)(xw, row) for s in range(NSLICE)]
    b1block = W1a[DX:]
    ts = []
    for s in range(NSLICE):
        ts.append(_mlp1(xwg[s], edge_attr[s * ES:(s + 1) * ES], b1block))
    g1r = g1.reshape(1, DH)
    be1r = be1.reshape(1, DH)
    b1br = b1b.reshape(1, DH)
    z128 = jnp.zeros((NP, DH), _F32)
    sp = []
    for s in range(NSLICE):
        y = _mlp1b(ts[s][0], ts[0][1], ts[1][1], g1r, be1r, W1b, b1br)
        sp.append(_scatter_kernel(s)(y, col, z128))
    return _node_mlp(sp[0], sp[1], cntp.reshape(NW, N), jnp.ones((NW, 1), _F32),
                     x, W2a[:DX], W2a[DX:], b2a.reshape(1, DH),
                     g2.reshape(1, DH), be2.reshape(1, DH), W2b,
                     b2b.reshape(1, DH))


# speedup vs baseline: 1.2305x; 1.2305x over previous
import functools

import jax
import jax.numpy as jnp
from jax import lax
from jax.experimental import pallas as pl
from jax.experimental.pallas import tpu as pltpu
from jax.experimental.pallas import tpu_sc as plsc

N = 10000
E = 320000
NSLICE = 1
ES = E // NSLICE
DX = 48
DH = 128
NC, NS = 2, 16
NW = NC * NS
EPW = ES // NW
CH = 400
NCHUNK = EPW // CH
SCH = 80
NSCH = EPW // SCH
CC = 2000
NP = 10240
STRIPE = NP // NS
_F32 = jnp.float32
_PREC = lax.Precision.DEFAULT


def _mesh():
    return plsc.VectorSubcoreMesh(core_axis_name="c", subcore_axis_name="s")


_SC_PARAMS = dict(
    mesh=_mesh(),
    compiler_params=pltpu.CompilerParams(needs_layout_passes=False),
)


def _count_body(col_hbm, cnt_hbm, colv, cnt_v):
    wid = lax.axis_index("s") * NC + lax.axis_index("c")
    base = wid * (E // NW)
    zeros16 = jnp.zeros((16,), _F32)
    ones16 = jnp.ones((16,), _F32)

    def zbody(i, carry):
        cnt_v[pl.ds(i * 16, 16)] = zeros16
        return carry

    lax.fori_loop(0, N // 16, zbody, 0)

    def body(k, carry):
        pltpu.sync_copy(col_hbm.at[pl.ds(base + k * CC, CC)], colv)

        def cbody(j, c):
            cvec = colv[pl.ds(j * 16, 16)]
            plsc.addupdate_scatter(cnt_v, [cvec], ones16)
            return c

        lax.fori_loop(0, CC // 16, cbody, 0)
        return carry

    lax.fori_loop(0, (E // NW) // CC, body, 0)
    pltpu.sync_copy(cnt_v, cnt_hbm.at[pl.ds(wid * N, N)])


@functools.cache
def _count_kernel():
    return pl.kernel(
        _count_body,
        out_type=jax.ShapeDtypeStruct((NW * N,), _F32),
        scratch_types=[
            pltpu.VMEM((CC,), jnp.int32),
            pltpu.VMEM((N,), _F32),
        ],
        **_SC_PARAMS,
    )


def _gather_body(sbase, xw_hbm, row_hbm, out_hbm,
                 idx0, idx1, rows0, rows1, sem_i0, sem_i1, sem_g0, sem_g1):
    wid = lax.axis_index("s") * NC + lax.axis_index("c")
    base = sbase + wid * EPW
    obase = wid * EPW
    sem_i = (sem_i0, sem_i1)
    sem_g = (sem_g0, sem_g1)
    idx_v = (idx0, idx1)
    rows_v = (rows0, rows1)

    def load(k, b):
        pltpu.async_copy(row_hbm.at[pl.ds(base + k * CH, CH)], idx_v[b], sem_i[b])

    def wait_load(k, b):
        pltpu.make_async_copy(row_hbm.at[pl.ds(base + k * CH, CH)], idx_v[b],
                              sem_i[b]).wait()

    load(0, 0)

    def step(k, b):
        wait_load(k, b)
        pltpu.async_copy(xw_hbm.at[idx_v[b]], rows_v[b], sem_g[b])

        @pl.when(k >= 1)
        def _():
            po = obase + (k - 1) * CH
            pltpu.make_async_copy(xw_hbm.at[idx_v[1 - b]], rows_v[1 - b],
                                  sem_g[1 - b]).wait()
            pltpu.sync_copy(rows_v[1 - b], out_hbm.at[pl.ds(po, CH)])

        @pl.when(k + 1 <= NCHUNK - 1)
        def _():
            load(k + 1, 1 - b)

    def pair(g, carry):
        step(2 * g, 0)
        step(2 * g + 1, 1)
        return carry

    lax.fori_loop(0, NCHUNK // 2, pair, 0)
    k_last = NCHUNK - 1
    wait_load(k_last, 0)
    pltpu.async_copy(xw_hbm.at[idx_v[0]], rows_v[0], sem_g[0])
    pltpu.make_async_copy(xw_hbm.at[idx_v[1]], rows_v[1], sem_g[1]).wait()
    pltpu.sync_copy(rows_v[1], out_hbm.at[pl.ds(obase + (k_last - 1) * CH, CH)])
    pltpu.make_async_copy(xw_hbm.at[idx_v[0]], rows_v[0], sem_g[0]).wait()
    pltpu.sync_copy(rows_v[0], out_hbm.at[pl.ds(obase + k_last * CH, CH)])


@functools.cache
def _gather_kernel(s):
    return pl.kernel(
        functools.partial(_gather_body, s * ES),
        out_type=jax.ShapeDtypeStruct((ES, DH), _F32),
        scratch_types=[
            pltpu.VMEM((CH,), jnp.int32),
            pltpu.VMEM((CH,), jnp.int32),
            pltpu.VMEM((CH, DH), _F32),
            pltpu.VMEM((CH, DH), _F32),
            pltpu.SemaphoreType.DMA,
            pltpu.SemaphoreType.DMA,
            pltpu.SemaphoreType.DMA,
            pltpu.SemaphoreType.DMA,
        ],
        **_SC_PARAMS,
    )


def _scatter_body(sbase, y_hbm, col_hbm, z128_hbm, out_s, tab_s, sidx0, sidx1,
                  ybuf0, ybuf1, sem_d0, sem_d1, sem_a0, sem_a1):
    cid = lax.axis_index("c")
    sid = lax.axis_index("s")
    wid = sid * NC + cid
    srow = sid * STRIPE
    sem_d = (sem_d0, sem_d1)
    sem_a = (sem_a0, sem_a1)
    idx_v = (sidx0, sidx1)
    ybuf = (ybuf0, ybuf1)
    pltpu.sync_copy(z128_hbm.at[pl.ds(srow, STRIPE)], tab_s.at[pl.ds(srow, STRIPE)])
    plsc.subcore_barrier()
    base = wid * EPW
    cbase = sbase + wid * EPW

    def load(k, b):
        pltpu.async_copy(col_hbm.at[pl.ds(cbase + k * SCH, SCH)], idx_v[b], sem_d[b])
        pltpu.async_copy(y_hbm.at[pl.ds(base + k * SCH, SCH)], ybuf[b], sem_d[b])

    def wait_load(k, b):
        pltpu.make_async_copy(col_hbm.at[pl.ds(cbase + k * SCH, SCH)], idx_v[b],
                              sem_d[b]).wait()
        pltpu.make_async_copy(y_hbm.at[pl.ds(base + k * SCH, SCH)], ybuf[b],
                              sem_d[b]).wait()

    def wait_add(b):
        pltpu.make_async_copy(ybuf[b], tab_s.at[idx_v[b]], sem_a[b]).wait()

    load(0, 0)
    load(1, 1)

    def step(k, b):
        wait_load(k, b)
        pltpu.async_copy(ybuf[b], tab_s.at[idx_v[b]], sem_a[b], add=True)

        @pl.when(jnp.logical_and(k >= 1, k + 1 <= NSCH - 1))
        def _():
            wait_add(1 - b)
            load(k + 1, 1 - b)

    def pair(g, carry):
        step(2 * g, 0)
        step(2 * g + 1, 1)
        return carry

    lax.fori_loop(0, NSCH // 2, pair, 0)
    wait_load(NSCH - 1, 0)
    pltpu.async_copy(ybuf[0], tab_s.at[idx_v[0]], sem_a[0], add=True)
    wait_add(1)
    wait_add(0)
    plsc.subcore_barrier()
    pltpu.sync_copy(tab_s.at[pl.ds(srow, STRIPE)],
                    out_s.at[pl.ds(cid * NP + srow, STRIPE)])


@functools.cache
def _scatter_kernel(s):
    return pl.kernel(
        functools.partial(_scatter_body, s * ES),
        out_type=jax.ShapeDtypeStruct((NC * NP, DH), _F32),
        scratch_types=[
            pltpu.VMEM_SHARED((NP, DH), _F32),
            pltpu.VMEM((SCH,), jnp.int32),
            pltpu.VMEM((SCH,), jnp.int32),
            pltpu.VMEM((SCH, DH), _F32),
            pltpu.VMEM((SCH, DH), _F32),
            pltpu.SemaphoreType.DMA,
            pltpu.SemaphoreType.DMA,
            pltpu.SemaphoreType.DMA,
            pltpu.SemaphoreType.DMA,
        ],
        **_SC_PARAMS,
    )


BE = 3200


def _xw_body(x_ref, a_ref, bias_ref, o_ref):
    o_ref[...] = (jnp.dot(x_ref[...], a_ref[...], preferred_element_type=_F32,
                          precision=_PREC) + bias_ref[...])


def _mlp1_body(xwg_ref, ea_ref, b_ref, t_ref, stat_ref):
    t = jnp.dot(ea_ref[...], b_ref[...], preferred_element_type=_F32,
                precision=_PREC)
    t = t + xwg_ref[...]
    t_ref[...] = t
    srow = jnp.sum(t, axis=0, keepdims=True)
    sqrow = jnp.sum(t * t, axis=0, keepdims=True)
    upd = jnp.concatenate([srow, sqrow, jnp.zeros((6, DH), _F32)], axis=0)

    @pl.when(pl.program_id(0) == 0)
    def _():
        stat_ref[...] = jnp.zeros_like(stat_ref)

    stat_ref[...] += upd


def _mlp1b_body(t_ref, stat_ref, g_ref, be_ref, w_ref, bias_ref, y_ref):
    stat = stat_ref[...]
    mean = stat[0:1, :] * (1.0 / E)
    var = stat[1:2, :] * (1.0 / E) - mean * mean
    scale = g_ref[...] * lax.rsqrt(var + 1e-5)
    shift = be_ref[...] - mean * scale
    tn = jnp.maximum(t_ref[...] * scale + shift, 0.0)
    y_ref[...] = (jnp.dot(tn, w_ref[...], preferred_element_type=_F32,
                          precision=_PREC) + bias_ref[...])


def _node_body(sa_ref, cp_ref, rones_ref, x_ref, a_ref, b_ref, b2a_ref,
               g_ref, be_ref, w_ref, b2b_ref, out_ref):
    ssum = sa_ref[0:N, :] + sa_ref[NP:NP + N, :]
    cnt = lax.dot_general(cp_ref[...], rones_ref[...], (((0,), (0,)), ((), ())),
                          preferred_element_type=_F32,
                          precision=lax.Precision.HIGHEST)
    agg = jnp.where(cnt > 0.0, ssum / jnp.maximum(cnt, 1.0), 0.0)
    h = (jnp.dot(x_ref[...], a_ref[...], preferred_element_type=_F32,
                 precision=_PREC)
         + jnp.dot(agg, b_ref[...], preferred_element_type=_F32, precision=_PREC)
         + b2a_ref[...])
    mean = jnp.mean(h, axis=0, keepdims=True)
    var = jnp.mean((h - mean) ** 2, axis=0, keepdims=True)
    hn = jnp.maximum((h - mean) * lax.rsqrt(var + 1e-5) * g_ref[...] + be_ref[...],
                     0.0)
    out_ref[...] = (jnp.dot(hn, w_ref[...], preferred_element_type=_F32,
                            precision=_PREC) + b2b_ref[...])


def _xw(x, a1, bias):
    return pl.pallas_call(
        _xw_body,
        out_shape=jax.ShapeDtypeStruct((N, DH), _F32),
    )(x, a1, bias)


def _mlp1(xwg, ea, b1):
    return pl.pallas_call(
        _mlp1_body,
        grid=(ES // BE,),
        in_specs=[
            pl.BlockSpec((BE, DH), lambda i: (i, 0)),
            pl.BlockSpec((BE, DH), lambda i: (i, 0)),
            pl.BlockSpec((DH, DH), lambda i: (0, 0)),
        ],
        out_specs=[
            pl.BlockSpec((BE, DH), lambda i: (i, 0)),
            pl.BlockSpec((8, DH), lambda i: (0, 0)),
        ],
        out_shape=[
            jax.ShapeDtypeStruct((ES, DH), _F32),
            jax.ShapeDtypeStruct((8, DH), _F32),
        ],
    )(xwg, ea, b1)


def _mlp1b(t, stats, g, be, w, bias):
    return pl.pallas_call(
        _mlp1b_body,
        grid=(ES // BE,),
        in_specs=[
            pl.BlockSpec((BE, DH), lambda i: (i, 0)),
            pl.BlockSpec((8, DH), lambda i: (0, 0)),
            pl.BlockSpec((1, DH), lambda i: (0, 0)),
            pl.BlockSpec((1, DH), lambda i: (0, 0)),
            pl.BlockSpec((DH, DH), lambda i: (0, 0)),
            pl.BlockSpec((1, DH), lambda i: (0, 0)),
        ],
        out_specs=pl.BlockSpec((BE, DH), lambda i: (i, 0)),
        out_shape=jax.ShapeDtypeStruct((ES, DH), _F32),
    )(t, stats, g, be, w, bias)


def _node_mlp(sa, cp, rones, x, a2, b2, b2a, g2, be2, w2b, b2b):
    return pl.pallas_call(
        _node_body,
        out_shape=jax.ShapeDtypeStruct((N, DH), _F32),
    )(sa, cp, rones, x, a2, b2, b2a, g2, be2, w2b, b2b)


def kernel(x, edge_index, edge_attr, u, batch, W1a, b1a, g1, be1, W1b, b1b,
           W2a, b2a, g2, be2, W2b, b2b):
    row = edge_index[0]
    col = edge_index[1]
    cntp = _count_kernel()(col)
    xw = _xw(x, W1a[:DX], b1a.reshape(1, DH))
    xwg = _gather_kernel(0)(xw, row)
    t, stats = _mlp1(xwg, edge_attr, W1a[DX:])
    y = _mlp1b(t, stats, g1.reshape(1, DH), be1.reshape(1, DH), W1b,
               b1b.reshape(1, DH))
    z128 = jnp.zeros((NP, DH), _F32)
    sp = _scatter_kernel(0)(y, col, z128)
    return _node_mlp(sp, cntp.reshape(NW, N), jnp.ones((NW, 1), _F32),
                     x, W2a[:DX], W2a[DX:], b2a.reshape(1, DH),
                     g2.reshape(1, DH), be2.reshape(1, DH), W2b,
                     b2b.reshape(1, DH))
